# Initial kernel scaffold; baseline (speedup 1.0000x reference)
import jax
import jax.numpy as jnp
from jax.experimental import pallas as pl


def kernel(x, edge_index, Wa0, ba0, Wp0, bp0, Wa1, ba1, Wp1, bp1, Wout, bout):
    raise NotImplementedError("write your pallas kernel here")


# speedup vs baseline: 4.0670x; 4.0670x over previous
import functools

import jax
import jax.numpy as jnp
from jax import lax
from jax.experimental import pallas as pl
from jax.experimental.pallas import tpu as pltpu
from jax.experimental.pallas import tpu_sc as plsc

N = 10000
E = 320000
D = 128
H = 128

N_PAD = 10240
NUM_SC = 2
TILES = 16
NW = NUM_SC * TILES
CHUNK = 128
EW = ((E + NW * CHUNK - 1) // (NW * CHUNK)) * CHUNK
E_PAD = EW * NW
N_CHUNKS = EW // CHUNK
RPT = N_PAD // TILES

BLK = 1024
GRID = N_PAD // BLK


def _elu(v):
    return jnp.where(v > 0, v, jnp.exp(jnp.minimum(v, 0.0)) - 1.0)


def _dot(a, w):
    return jnp.dot(a.astype(jnp.bfloat16), w.astype(jnp.bfloat16),
                   preferred_element_type=jnp.float32)


def _stage1_body(x_ref, w_ref, b_ref, o_ref):
    o_ref[...] = _elu(_dot(x_ref[...], w_ref[...]) + b_ref[...])


def _stage3_body(p0_ref, p1_ref, wp_ref, bp_ref, wa_ref, ba_ref,
                 h0_ref, y1_ref):
    agg = p0_ref[...] + p1_ref[...]
    h0 = _elu(_dot(agg, wp_ref[...]) + bp_ref[...])
    h0_ref[...] = h0
    y1_ref[...] = _elu(_dot(h0, wa_ref[...]) + ba_ref[...])


def _stage5_body(q0_ref, q1_ref, h0_ref, wpa_ref, wpb_ref, bp_ref,
                 wo_ref, bo_ref, s_ref):
    agg = q0_ref[...] + q1_ref[...]
    h1 = _elu(_dot(agg, wpa_ref[...]) + _dot(h0_ref[...], wpb_ref[...])
              + bp_ref[...])
    h1b = h1.astype(jnp.bfloat16).astype(jnp.float32)
    wob = wo_ref[...].astype(jnp.bfloat16).astype(jnp.float32)
    s_ref[...] = jnp.sum(h1b * wob, axis=1) + bo_ref[0, 0]


_row_spec = pl.BlockSpec((BLK, D), lambda i: (i, 0))
_w_spec = pl.BlockSpec((D, H), lambda i: (0, 0))
_b_spec = pl.BlockSpec((1, H), lambda i: (0, 0))

_stage1 = pl.pallas_call(
    _stage1_body,
    grid=(GRID,),
    in_specs=[_row_spec, _w_spec, _b_spec],
    out_specs=_row_spec,
    out_shape=jax.ShapeDtypeStruct((N_PAD, H), jnp.float32),
)

_stage3 = pl.pallas_call(
    _stage3_body,
    grid=(GRID,),
    in_specs=[_row_spec, _row_spec, _w_spec, _b_spec, _w_spec, _b_spec],
    out_specs=[_row_spec, _row_spec],
    out_shape=[jax.ShapeDtypeStruct((N_PAD, H), jnp.float32),
               jax.ShapeDtypeStruct((N_PAD, H), jnp.float32)],
)

_stage5 = pl.pallas_call(
    _stage5_body,
    grid=(GRID,),
    in_specs=[_row_spec, _row_spec, _row_spec, _w_spec, _w_spec, _b_spec,
              _b_spec, pl.BlockSpec((1, 1), lambda i: (0, 0))],
    out_specs=pl.BlockSpec((BLK,), lambda i: (i,)),
    out_shape=jax.ShapeDtypeStruct((N_PAD,), jnp.float32),
)


@functools.cache
def _make_sc_segment_sum():
    @functools.partial(
        pl.kernel,
        out_type=jax.ShapeDtypeStruct((NUM_SC, N_PAD, D), jnp.float32),
        mesh=plsc.VectorSubcoreMesh(core_axis_name="c", subcore_axis_name="s"),
        scratch_types=[
            pltpu.VMEM((CHUNK,), jnp.int32),
            pltpu.VMEM((CHUNK,), jnp.int32),
            pltpu.VMEM((CHUNK, D), jnp.float32),
            pltpu.VMEM_SHARED((N_PAD, D), jnp.float32),
            pltpu.SemaphoreType.DMA,
        ],
    )
    def _sc_segment_sum(y_hbm, src_hbm, dst_hbm, zeros_hbm, parts_hbm,
                        src_v, dst_v, rows_v, acc_sh, sem):
        c = lax.axis_index("c")
        s = lax.axis_index("s")
        pltpu.sync_copy(zeros_hbm, acc_sh.at[pl.ds(s * RPT, RPT)])
        plsc.subcore_barrier()

        base = (c * TILES + s) * EW

        def body(i, carry):
            off = base + i * CHUNK
            pltpu.sync_copy(src_hbm.at[pl.ds(off, CHUNK)], src_v)
            pltpu.sync_copy(dst_hbm.at[pl.ds(off, CHUNK)], dst_v)
            pltpu.async_copy(y_hbm.at[src_v], rows_v, sem).wait()
            pltpu.sync_copy(rows_v, acc_sh.at[dst_v], add=True)
            return carry

        lax.fori_loop(0, N_CHUNKS, body, 0)
        plsc.subcore_barrier()
        pltpu.sync_copy(acc_sh.at[pl.ds(s * RPT, RPT)],
                        parts_hbm.at[c, pl.ds(s * RPT, RPT)])

    return _sc_segment_sum


def kernel(x, edge_index, Wa0, ba0, Wp0, bp0, Wa1, ba1, Wp1, bp1, Wout, bout):
    x_pad = jnp.zeros((N_PAD, D), jnp.float32).at[:N].set(x)
    pad = jnp.full((E_PAD - E,), N, dtype=jnp.int32)
    src = jnp.concatenate([edge_index[0], pad])
    dst = jnp.concatenate([edge_index[1], pad])
    zeros = jnp.zeros((RPT, D), jnp.float32)

    ba0r = ba0.reshape(1, H)
    bp0r = bp0.reshape(1, H)
    ba1r = ba1.reshape(1, H)
    bp1r = bp1.reshape(1, H)
    wor = Wout.reshape(1, H)
    bor = bout.reshape(1, 1)

    sc_segment_sum = _make_sc_segment_sum()
    y0 = _stage1(x_pad, Wa0, ba0r)
    parts0 = sc_segment_sum(y0, src, dst, zeros)
    h0, y1 = _stage3(parts0[0], parts0[1], Wp0, bp0r, Wa1, ba1r)
    parts1 = sc_segment_sum(y1, src, dst, zeros)
    scores = _stage5(parts1[0], parts1[1], h0, Wp1[:H], Wp1[H:], bp1r,
                     wor, bor)
    return scores[:N]
